# baseline (speedup 1.0000x reference)
import functools

import jax
import jax.numpy as jnp
from jax import lax
from jax.experimental import pallas as pl
from jax.experimental.pallas import tpu as pltpu
from jax.experimental.pallas import tpu_sc as plsc

_B, _S = 16384, 50
_D = 64
_NC, _NS = 2, 16
_NW = _NC * _NS
_RPW = _B // _NW
_NBUF = 8
_H = 4


def _sc_gather(idx, table):
    mesh = plsc.VectorSubcoreMesh(core_axis_name="c", subcore_axis_name="s")

    @functools.partial(
        pl.kernel,
        mesh=mesh,
        out_type=jax.ShapeDtypeStruct((_B, _S, _D), jnp.float32),
        compiler_params=pltpu.CompilerParams(use_tc_tiling_on_sc=False),
        scratch_types=[
            pltpu.VMEM((_RPW, _S), jnp.int32),
            pltpu.VMEM((_NBUF, _S, _D), jnp.float32),
            [pltpu.SemaphoreType.DMA] * _NBUF,
            [pltpu.SemaphoreType.DMA] * _NBUF,
        ],
    )
    def k(idx_hbm, table_hbm, out_hbm, idx_v, rows_v, sem_g, sem_s):
        wid = lax.axis_index("s") * _NC + lax.axis_index("c")
        base = wid * _RPW
        pltpu.sync_copy(idx_hbm.at[pl.ds(base, _RPW)], idx_v)

        def fire_gather(r, b):
            pltpu.async_copy(table_hbm.at[idx_v.at[r]], rows_v.at[b], sem_g[b])

        def wait_gather(b):
            pltpu.make_async_copy(
                table_hbm.at[idx_v.at[0]], rows_v.at[b], sem_g[b]
            ).wait()

        def fire_store(r, b):
            pltpu.async_copy(rows_v.at[b], out_hbm.at[base + r], sem_s[b])

        def wait_store(b):
            pltpu.make_async_copy(
                rows_v.at[b], out_hbm.at[base], sem_s[b]
            ).wait()

        for b in range(_H):
            fire_gather(b, b)

        def body(g, _):
            r0 = g * _NBUF
            for b in range(_NBUF):
                r = r0 + b
                f = r + _H
                bf = (b + _H) % _NBUF

                @pl.when(f < _RPW)
                def _():
                    @pl.when(f >= _NBUF)
                    def _():
                        wait_store(bf)
                    fire_gather(f, bf)

                wait_gather(b)
                fire_store(r, b)
            return 0

        lax.fori_loop(0, _RPW // _NBUF, body, 0)

        for b in range(_NBUF - _H, _NBUF):
            wait_store(b)

    return k(idx, table)


def kernel(x, table):
    return _sc_gather(x.astype(jnp.int32), table)


# speedup vs baseline: 1.0048x; 1.0048x over previous
import functools

import jax
import jax.numpy as jnp
from jax import lax
from jax.experimental import pallas as pl
from jax.experimental.pallas import tpu as pltpu
from jax.experimental.pallas import tpu_sc as plsc

_B, _S = 16384, 50
_N = _B * _S
_D = 64
_NC, _NS = 2, 16
_NW = _NC * _NS
_BPW = _N // _NW
_CH = 128
_NCH = _BPW // _CH
_NBUF = 8
_H = 4


def _sc_gather(idx, table):
    mesh = plsc.VectorSubcoreMesh(core_axis_name="c", subcore_axis_name="s")

    @functools.partial(
        pl.kernel,
        mesh=mesh,
        out_type=jax.ShapeDtypeStruct((_N, _D), jnp.float32),
        compiler_params=pltpu.CompilerParams(use_tc_tiling_on_sc=False),
        scratch_types=[
            pltpu.VMEM((_NCH, _CH), jnp.int32),
            pltpu.VMEM((_NBUF, _CH, _D), jnp.float32),
            [pltpu.SemaphoreType.DMA] * _NBUF,
            [pltpu.SemaphoreType.DMA] * _NBUF,
        ],
    )
    def k(idx_hbm, table_hbm, out_hbm, idx_v, rows_v, sem_g, sem_s):
        wid = lax.axis_index("s") * _NC + lax.axis_index("c")
        base = wid * _BPW
        pltpu.sync_copy(idx_hbm.at[pl.ds(wid * _NCH, _NCH)], idx_v)

        def fire_gather(j, b):
            pltpu.async_copy(table_hbm.at[idx_v.at[j]], rows_v.at[b], sem_g[b])

        def wait_gather(b):
            pltpu.make_async_copy(
                table_hbm.at[idx_v.at[0]], rows_v.at[b], sem_g[b]
            ).wait()

        def fire_store(j, b):
            pltpu.async_copy(
                rows_v.at[b], out_hbm.at[pl.ds(base + j * _CH, _CH)], sem_s[b]
            )

        def wait_store(b):
            pltpu.make_async_copy(
                rows_v.at[b], out_hbm.at[pl.ds(base, _CH)], sem_s[b]
            ).wait()

        for b in range(_H):
            fire_gather(b, b)

        def body(g, _):
            j0 = g * _NBUF
            for b in range(_NBUF):
                j = j0 + b
                f = j + _H
                bf = (b + _H) % _NBUF

                @pl.when(f < _NCH)
                def _():
                    @pl.when(f >= _NBUF)
                    def _():
                        wait_store(bf)
                    fire_gather(f, bf)

                wait_gather(b)
                fire_store(j, b)
            return 0

        lax.fori_loop(0, _NCH // _NBUF, body, 0)

        for b in range(_NBUF - _H, _NBUF):
            wait_store(b)

    return k(idx, table)


def kernel(x, table):
    idx = x.astype(jnp.int32).reshape(_N // _CH, _CH)
    out = _sc_gather(idx, table)
    return out.reshape(_B, _S, _D)


# speedup vs baseline: 1.3479x; 1.3415x over previous
import functools

import jax
import jax.numpy as jnp
from jax import lax
from jax.experimental import pallas as pl
from jax.experimental.pallas import tpu as pltpu
from jax.experimental.pallas import tpu_sc as plsc

_B, _S = 16384, 50
_SP = 56
_D = 64
_DP = 128
_NC, _NS = 2, 16
_NW = _NC * _NS
_RPW = _B // _NW
_NBUF = 8
_H = 4


def _sc_gather(idx, table):
    mesh = plsc.VectorSubcoreMesh(core_axis_name="c", subcore_axis_name="s")

    @functools.partial(
        pl.kernel,
        mesh=mesh,
        out_type=jax.ShapeDtypeStruct((_B, _SP, _DP), jnp.float32),
        compiler_params=pltpu.CompilerParams(use_tc_tiling_on_sc=False),
        scratch_types=[
            pltpu.VMEM((_RPW, _S), jnp.int32),
            pltpu.VMEM((_NBUF, _S, _D), jnp.float32),
            [pltpu.SemaphoreType.DMA] * _NBUF,
            [pltpu.SemaphoreType.DMA] * _NBUF,
        ],
    )
    def k(idx_hbm, table_hbm, out_hbm, idx_v, rows_v, sem_g, sem_s):
        wid = lax.axis_index("s") * _NC + lax.axis_index("c")
        base = wid * _RPW
        pltpu.sync_copy(idx_hbm.at[pl.ds(base, _RPW)], idx_v)

        def fire_gather(r, b):
            pltpu.async_copy(table_hbm.at[idx_v.at[r]], rows_v.at[b], sem_g[b])

        def wait_gather(b):
            pltpu.make_async_copy(
                table_hbm.at[idx_v.at[0]], rows_v.at[b], sem_g[b]
            ).wait()

        def fire_store(r, b):
            pltpu.async_copy(
                rows_v.at[b],
                out_hbm.at[base + r, pl.ds(0, _S), pl.ds(0, _D)],
                sem_s[b],
            )

        def wait_store(b):
            pltpu.make_async_copy(
                rows_v.at[b],
                out_hbm.at[base, pl.ds(0, _S), pl.ds(0, _D)],
                sem_s[b],
            ).wait()

        for b in range(_H):
            fire_gather(b, b)

        def body(g, _):
            r0 = g * _NBUF
            for b in range(_NBUF):
                r = r0 + b
                f = r + _H
                bf = (b + _H) % _NBUF

                @pl.when(f < _RPW)
                def _():
                    @pl.when(f >= _NBUF)
                    def _():
                        wait_store(bf)
                    fire_gather(f, bf)

                wait_gather(b)
                fire_store(r, b)
            return 0

        lax.fori_loop(0, _RPW // _NBUF, body, 0)

        for b in range(_NBUF - _H, _NBUF):
            wait_store(b)

    return k(idx, table)


def kernel(x, table):
    out3 = _sc_gather(x.astype(jnp.int32), table)
    return out3[:, :_S, :_D]
